# baseline (speedup 1.0000x reference)
import jax
import jax.numpy as jnp
from jax import lax
from jax.experimental import pallas as pl
from jax.experimental.pallas import tpu as pltpu
from jax.experimental.pallas import tpu_sc as plsc

EMBED_DIM = 128


def _scs_lookup(idx_hbm, table_hbm, out_hbm, idx_s):
    pltpu.sync_copy(idx_hbm, idx_s)
    i = idx_s[0]
    pltpu.sync_copy(table_hbm.at[pl.ds(i, 1)], out_hbm)


def kernel(weight, table):
    idx = jnp.asarray(weight, dtype=jnp.int32).reshape(1)
    mesh = plsc.ScalarSubcoreMesh(axis_name="c", num_cores=1)
    out = pl.kernel(
        _scs_lookup,
        out_type=jax.ShapeDtypeStruct((1, EMBED_DIM), jnp.float32),
        mesh=mesh,
        scratch_types=[
            pltpu.SMEM((1,), jnp.int32),
        ],
    )(idx, table)
    return out[0]


# speedup vs baseline: 1.0236x; 1.0236x over previous
import jax
import jax.numpy as jnp
from jax.experimental import pallas as pl
from jax.experimental.pallas import tpu as pltpu
from jax.experimental.pallas import tpu_sc as plsc

EMBED_DIM = 128


def _scs_lookup(idx_hbm, table_hbm, out_hbm, idx_s):
    pltpu.sync_copy(idx_hbm, idx_s)
    i = idx_s[0]
    pltpu.sync_copy(table_hbm.at[pl.ds(i, 1)], out_hbm)


def kernel(weight, table):
    idx = jnp.asarray(weight, dtype=jnp.int32).reshape(1)
    mesh = plsc.ScalarSubcoreMesh(axis_name="c", num_cores=1)
    out = pl.kernel(
        _scs_lookup,
        out_type=jax.ShapeDtypeStruct((1, EMBED_DIM), jnp.float32),
        mesh=mesh,
        scratch_types=[
            pltpu.SMEM((1,), jnp.int32),
        ],
    )(idx, table)
    return out[0]
